# baseline (speedup 1.0000x reference)
import functools

import jax
import jax.numpy as jnp
from jax import lax
from jax.experimental import pallas as pl
from jax.experimental.pallas import tpu as pltpu
from jax.experimental.pallas import tpu_sc as plsc

N_BINS = 15
_C = 128


def _dense_kernel(logits_ref, labels_ref, conf_ref, acc_ref):
    x = logits_ref[...]
    lab = labels_ref[...]
    m = jnp.max(x, axis=1, keepdims=True)
    e = jnp.exp(x - m)
    s = jnp.sum(e, axis=1, keepdims=True)
    conf = 1.0 / s
    lane = jax.lax.broadcasted_iota(jnp.int32, x.shape, 1)
    e_lab = jnp.sum(jnp.where(lane == lab, e, 0.0), axis=1, keepdims=True)
    conf_ref[...] = conf
    acc_ref[...] = jnp.floor(e_lab)


def _make_sc_seg(n, nw):
    shard = n // nw
    mesh = plsc.VectorSubcoreMesh(core_axis_name="c", subcore_axis_name="s")

    @functools.partial(
        pl.kernel,
        mesh=mesh,
        out_type=[
            jax.ShapeDtypeStruct((nw, 240), jnp.float32),
            jax.ShapeDtypeStruct((nw, 240), jnp.float32),
            jax.ShapeDtypeStruct((nw, 240), jnp.float32),
        ],
        scratch_types=[
            pltpu.VMEM((shard,), jnp.float32),
            pltpu.VMEM((shard,), jnp.float32),
            pltpu.VMEM((240,), jnp.float32),
            pltpu.VMEM((240,), jnp.float32),
            pltpu.VMEM((240,), jnp.float32),
        ],
    )
    def sc_seg(conf_hbm, acc_hbm, o_cnt, o_acc, o_conf,
               conf_v, acc_v, h0, h1, h2):
        wid = lax.axis_index("s") * 2 + lax.axis_index("c")
        base = wid * shard
        pltpu.sync_copy(conf_hbm.at[pl.ds(base, shard)], conf_v)
        pltpu.sync_copy(acc_hbm.at[pl.ds(base, shard)], acc_v)
        zero16 = jnp.zeros((16,), jnp.float32)
        ones16 = jnp.ones((16,), jnp.float32)
        nbins16 = jnp.full((16,), float(N_BINS), jnp.float32)
        maxbin16 = jnp.full((16,), N_BINS - 1, jnp.int32)
        binvecs = [jnp.full((16,), b, jnp.int32) for b in range(N_BINS)]
        for b in range(N_BINS):
            h0[pl.ds(b * 16, 16)] = zero16
            h1[pl.ds(b * 16, 16)] = zero16
            h2[pl.ds(b * 16, 16)] = zero16

        def body(i, carry):
            c = conf_v[pl.ds(i * 16, 16)]
            a = acc_v[pl.ds(i * 16, 16)]
            bi = jnp.minimum((c * nbins16).astype(jnp.int32), maxbin16)
            for b in range(N_BINS):
                mask = bi == binvecs[b]
                plsc.addupdate(h0.at[pl.ds(b * 16, 16)], jnp.where(mask, ones16, zero16))
                plsc.addupdate(h1.at[pl.ds(b * 16, 16)], jnp.where(mask, a, zero16))
                plsc.addupdate(h2.at[pl.ds(b * 16, 16)], jnp.where(mask, c, zero16))
            return carry

        lax.fori_loop(0, shard // 16, body, jnp.zeros((16,), jnp.float32))
        pltpu.sync_copy(h0, o_cnt.at[wid])
        pltpu.sync_copy(h1, o_acc.at[wid])
        pltpu.sync_copy(h2, o_conf.at[wid])

    return sc_seg


def _combine_kernel(cnt_ref, sacc_ref, sconf_ref, bin_ece_ref, ece_ref, *, n_total):
    count = jnp.sum(cnt_ref[...], axis=1, keepdims=True)
    sum_acc = jnp.sum(sacc_ref[...], axis=1, keepdims=True)
    sum_conf = jnp.sum(sconf_ref[...], axis=1, keepdims=True)
    safe = jnp.maximum(count, 1.0)
    prop = count / float(n_total)
    bin_ece = jnp.where(
        count > 0.0, jnp.abs(sum_conf / safe - sum_acc / safe) * prop, 0.0
    )
    bin_ece_ref[...] = bin_ece
    ece_ref[...] = jnp.sum(bin_ece, keepdims=True)


def kernel(logits, labels):
    n, c = logits.shape
    blk = 8192
    grid = n // blk
    labels2d = labels.reshape(n, 1)
    conf2d, acc2d = pl.pallas_call(
        _dense_kernel,
        grid=(grid,),
        in_specs=[
            pl.BlockSpec((blk, c), lambda i: (i, 0)),
            pl.BlockSpec((blk, 1), lambda i: (i, 0)),
        ],
        out_specs=[
            pl.BlockSpec((blk, 1), lambda i: (i, 0)),
            pl.BlockSpec((blk, 1), lambda i: (i, 0)),
        ],
        out_shape=[
            jax.ShapeDtypeStruct((n, 1), jnp.float32),
            jax.ShapeDtypeStruct((n, 1), jnp.float32),
        ],
        compiler_params=pltpu.CompilerParams(
            dimension_semantics=("arbitrary",),
        ),
    )(logits, labels2d)

    nw = 32
    cnt, sacc, sconf = _make_sc_seg(n, nw)(
        conf2d.reshape(n), acc2d.reshape(n)
    )
    tr = lambda a: a.reshape(nw, N_BINS, 16).transpose(1, 0, 2).reshape(N_BINS, nw * 16)

    bin_ece2d, ece = pl.pallas_call(
        functools.partial(_combine_kernel, n_total=n),
        out_shape=[
            jax.ShapeDtypeStruct((N_BINS, 1), jnp.float32),
            jax.ShapeDtypeStruct((1, 1), jnp.float32),
        ],
    )(tr(cnt), tr(sacc), tr(sconf))
    return ece[0, 0], bin_ece2d[:, 0]


# speedup vs baseline: 1.6952x; 1.6952x over previous
import functools

import jax
import jax.numpy as jnp
from jax.experimental import pallas as pl
from jax.experimental.pallas import tpu as pltpu

N_BINS = 15
_C = 128


def _ece_tc_kernel(logits_ref, labels_ref, bin_ece_ref, ece_ref, acc_ref, *, n_total):
    i = pl.program_id(0)
    nsteps = pl.num_programs(0)

    @pl.when(i == 0)
    def _init():
        acc_ref[...] = jnp.zeros_like(acc_ref)

    x = logits_ref[...]
    lab = labels_ref[...]
    m = jnp.max(x, axis=1, keepdims=True)
    e = jnp.exp(x - m)
    s = jnp.sum(e, axis=1, keepdims=True)
    conf = 1.0 / s

    lane = jax.lax.broadcasted_iota(jnp.int32, x.shape, 1)
    e_lab = jnp.sum(jnp.where(lane == lab, e, 0.0), axis=1, keepdims=True)
    acc = jnp.floor(e_lab)

    lane_f = jax.lax.broadcasted_iota(jnp.int32, (1, _C), 1).astype(jnp.float32)
    cmpb = (conf > lane_f * (1.0 / N_BINS)) & (
        conf <= (lane_f + 1.0) * (1.0 / N_BINS)
    )
    combo = jnp.where(cmpb, 4096.0 + acc, 0.0)
    confv = jnp.where(cmpb, conf, 0.0)
    nslc = x.shape[0] // 2048
    combo4 = jnp.sum(combo.reshape(nslc, 2048, _C), axis=1)
    conf_s = jnp.sum(confv, axis=0, keepdims=True)
    cnt4 = jnp.floor(combo4 * (1.0 / 4096.0))
    acc_ref[0:1, :] += jnp.sum(cnt4, axis=0, keepdims=True)
    acc_ref[1:2, :] += jnp.sum(combo4 - 4096.0 * cnt4, axis=0, keepdims=True)
    acc_ref[2:3, :] += conf_s

    del nsteps
    count = acc_ref[0:1, :]
    sum_acc = acc_ref[1:2, :]
    sum_conf = acc_ref[2:3, :]
    safe = jnp.maximum(count, 1.0)
    prop = count / float(n_total)
    bin_ece = jnp.where(
        count > 0.0, jnp.abs(sum_conf / safe - sum_acc / safe) * prop, 0.0
    )
    bin_ece_ref[...] = bin_ece
    ece_ref[...] = jnp.sum(bin_ece, keepdims=True)


def kernel(logits, labels):
    n, c = logits.shape
    blk = 8192
    grid = n // blk
    labels2d = labels.reshape(n, 1)
    bin_ece_pad, ece = pl.pallas_call(
        functools.partial(_ece_tc_kernel, n_total=n),
        grid=(grid,),
        in_specs=[
            pl.BlockSpec((blk, c), lambda i: (i, 0)),
            pl.BlockSpec((blk, 1), lambda i: (i, 0)),
        ],
        out_specs=[
            pl.BlockSpec((1, 128), lambda i: (0, 0)),
            pl.BlockSpec((1, 1), lambda i: (0, 0)),
        ],
        out_shape=[
            jax.ShapeDtypeStruct((1, 128), jnp.float32),
            jax.ShapeDtypeStruct((1, 1), jnp.float32),
        ],
        scratch_shapes=[pltpu.VMEM((8, 128), jnp.float32)],
        compiler_params=pltpu.CompilerParams(
            dimension_semantics=("arbitrary",),
        ),
    )(logits, labels2d)
    return ece[0, 0], bin_ece_pad[0, :N_BINS]
